# baseline (speedup 1.0000x reference)
import jax
import jax.numpy as jnp
from jax import lax
from jax.experimental import pallas as pl
from jax.experimental.pallas import tpu as pltpu
from jax.experimental.pallas import tpu_sc as plsc

_VB = 3584


def _sc_gather(ids_hbm, table_hbm, out_hbm, idx_v, rows_v, sem):
    c = lax.axis_index("c")
    s = lax.axis_index("s")

    @pl.when(jnp.logical_and(c == 0, s == 0))
    def _():
        pltpu.sync_copy(ids_hbm, idx_v)
        pltpu.async_copy(table_hbm.at[idx_v], rows_v, sem).wait()
        pltpu.sync_copy(rows_v, out_hbm)


def _mm_kernel(x_ref, w_ref, out_ref):
    res = jax.lax.dot_general(
        x_ref[...],
        w_ref[...],
        dimension_numbers=(((1,), (1,)), ((), ())),
        preferred_element_type=jnp.float32,
    )
    out_ref[...] = res[:, None, :]


def kernel(input_ids, emb_table, W_out):
    B, S = input_ids.shape
    V, D = W_out.shape
    n = B * S
    ids = input_ids.reshape(-1).astype(jnp.int32)

    mesh = plsc.VectorSubcoreMesh(core_axis_name="c", subcore_axis_name="s")
    gather = pl.kernel(
        _sc_gather,
        out_type=jax.ShapeDtypeStruct((n, D), jnp.float32),
        mesh=mesh,
        scratch_types=[
            pltpu.VMEM((n,), jnp.int32),
            pltpu.VMEM((n, D), jnp.float32),
            pltpu.SemaphoreType.DMA,
        ],
    )
    x = gather(ids, emb_table)

    nv = pl.cdiv(V, _VB)
    out = pl.pallas_call(
        _mm_kernel,
        grid=(nv,),
        in_specs=[
            pl.BlockSpec((n, D), lambda v: (0, 0)),
            pl.BlockSpec((_VB, D), lambda v: (v, 0)),
        ],
        out_specs=pl.BlockSpec((B, S, _VB), lambda v: (0, 0, v)),
        out_shape=jax.ShapeDtypeStruct((B, S, V), jnp.float32),
        compiler_params=pltpu.CompilerParams(
            dimension_semantics=("arbitrary",),
        ),
    )(x, W_out)
    return out


# speedup vs baseline: 1.1898x; 1.1898x over previous
import jax
import jax.numpy as jnp
from jax.experimental import pallas as pl
from jax.experimental.pallas import tpu as pltpu

_VB = 3584


def _lm_head_kernel(ids_ref, emb_hbm, w_ref, out_ref, x_ref, sem):
    nb = x_ref.shape[0]

    @pl.when(pl.program_id(0) == 0)
    def _gather():
        for b in range(nb):
            pltpu.make_async_copy(
                emb_hbm.at[pl.ds(ids_ref[b], 1), :],
                x_ref.at[pl.ds(b, 1), :],
                sem,
            ).start()
        for b in range(nb):
            pltpu.make_async_copy(
                emb_hbm.at[pl.ds(ids_ref[b], 1), :],
                x_ref.at[pl.ds(b, 1), :],
                sem,
            ).wait()

    res = jax.lax.dot_general(
        x_ref[...],
        w_ref[...],
        dimension_numbers=(((1,), (1,)), ((), ())),
        preferred_element_type=jnp.float32,
    )
    out_ref[...] = res[:, None, :]


def kernel(input_ids, emb_table, W_out):
    B, S = input_ids.shape
    V, D = W_out.shape
    ids = input_ids.reshape(-1).astype(jnp.int32)
    nv = pl.cdiv(V, _VB)
    out = pl.pallas_call(
        _lm_head_kernel,
        grid_spec=pltpu.PrefetchScalarGridSpec(
            num_scalar_prefetch=1,
            grid=(nv,),
            in_specs=[
                pl.BlockSpec(memory_space=pltpu.MemorySpace.HBM),
                pl.BlockSpec((_VB, D), lambda v, ids: (v, 0)),
            ],
            out_specs=pl.BlockSpec((B, S, _VB), lambda v, ids: (0, 0, v)),
            scratch_shapes=[
                pltpu.VMEM((B * S, D), jnp.float32),
                pltpu.SemaphoreType.DMA,
            ],
        ),
        out_shape=jax.ShapeDtypeStruct((B, S, V), jnp.float32),
        compiler_params=pltpu.CompilerParams(
            dimension_semantics=("arbitrary",),
        ),
    )(ids, emb_table, W_out)
    return out
